# baseline (speedup 1.0000x reference)
import functools
import numpy as np
import jax
import jax.numpy as jnp
from jax import lax
from jax.experimental import pallas as pl
from jax.experimental.pallas import tpu as pltpu

_F = 1024
_NS = 8
_N = 1024 * _NS
_R = 64
_G = _N // _R

_U32 = np.uint32


def _np_rotl(x, r):
    return ((x << _U32(r)) | (x >> _U32(32 - r))).astype(_U32)


def _np_threefry(k1, k2, x0, x1):
    ks0, ks1 = _U32(k1), _U32(k2)
    ks2 = _U32(ks0 ^ ks1 ^ _U32(0x1BD11BDA))
    x0 = (x0 + ks0).astype(_U32)
    x1 = (x1 + ks1).astype(_U32)
    rot = ([13, 15, 26, 6], [17, 29, 16, 24])
    inj = [(ks1, ks2, 1), (ks2, ks0, 2), (ks0, ks1, 3), (ks1, ks2, 4),
           (ks2, ks0, 5)]
    for i in range(5):
        for r in rot[i % 2]:
            x0 = (x0 + x1).astype(_U32)
            x1 = _np_rotl(x1, r)
            x1 = (x1 ^ x0).astype(_U32)
        a, b, c = inj[i]
        x0 = (x0 + a).astype(_U32)
        x1 = (x1 + b + _U32(c)).astype(_U32)
    return x0, x1


_kc1, _kc2 = _np_threefry(0, 42, _U32(0), _U32(0))
_kp1, _kp2 = _np_threefry(0, 42, _U32(0), _U32(1))

_TINY = np.float32(np.finfo(np.float32).tiny)


def _const(v):
    return jnp.int32(np.uint32(v).view(np.int32))


def _rotl(x, r):
    return (x << r) | lax.shift_right_logical(x, 32 - r)


def _threefry(k1, k2, x0, x1):
    ks0 = _const(k1) if isinstance(k1, (int, np.integer)) else k1
    ks1 = _const(k2) if isinstance(k2, (int, np.integer)) else k2
    ks2 = ks0 ^ ks1 ^ _const(0x1BD11BDA)
    x0 = x0 + ks0
    x1 = x1 + ks1
    rot = ([13, 15, 26, 6], [17, 29, 16, 24])
    inj = [(ks1, ks2, 1), (ks2, ks0, 2), (ks0, ks1, 3), (ks1, ks2, 4),
           (ks2, ks0, 5)]
    for i in range(5):
        for r in rot[i % 2]:
            x0 = x0 + x1
            x1 = _rotl(x1, r)
            x1 = x1 ^ x0
        a, b, c = inj[i]
        x0 = x0 + a
        x1 = x1 + b + _const(c)
    return x0, x1


def _uniform_gumbel(bits):
    fb = lax.shift_right_logical(bits, 9) | _const(0x3F800000)
    f = lax.bitcast_convert_type(fb, jnp.float32) - jnp.float32(1.0)
    u = jnp.maximum(jnp.float32(_TINY), f + jnp.float32(_TINY))
    return -jnp.log(-jnp.log(u))


def _body(logits_ref, perm_ref, out_ref):
    g = pl.program_id(0)
    rows = g * _R + lax.broadcasted_iota(jnp.int32, (_R, 1), 0)

    c = lax.broadcasted_iota(jnp.int32, (_R, _F), 1)
    idx = rows * (_F - 1) + c
    y1, y2 = _threefry(int(_kc1), int(_kc2), jnp.zeros((_R, _F), jnp.int32), idx)
    gum = _uniform_gumbel(y1 ^ y2)
    score = gum + logits_ref[0, :][None, :]
    m = jnp.max(score, axis=1, keepdims=True)
    ni = jnp.min(jnp.where(score == m, c, _F), axis=1, keepdims=True)
    k = ni + 1

    z1 = jnp.zeros((_R, 1), jnp.int32)
    pk1, pk2 = _threefry(int(_kp1), int(_kp2), z1, rows)
    s1, s2 = _threefry(pk1, pk2, z1, z1 + 1)
    cphys = lax.broadcasted_iota(jnp.int32, (_R, _F), 1)
    jlab = ((cphys & 127) << 3) | lax.shift_right_logical(cphys, 7)
    b1, b2 = _threefry(jnp.broadcast_to(s1, (_R, _F)),
                       jnp.broadcast_to(s2, (_R, _F)),
                       jnp.zeros((_R, _F), jnp.int32), jlab)
    bits = b1 ^ b2
    payload = (jlab < k).astype(jnp.int32)
    key = ((bits ^ _const(0x80000000)) & _const(0xFFFFFFFE)) | payload

    crow = cphys[0:1, :]
    lrow = jlab[0:1, :]

    def _flip_mask(size):
        return 0 - ((lrow & size) != 0).astype(jnp.int32)

    x = key ^ _flip_mask(2)
    for size_log in range(1, 11):
        size = 1 << size_log
        for s_log in range(size_log - 1, -1, -1):
            s = 1 << s_log
            d = s * 128 if s < 8 else s // 8
            lower = (crow & d) == 0
            x = jnp.where(lower,
                          jnp.minimum(x, jnp.roll(x, -d, axis=1)),
                          jnp.maximum(x, jnp.roll(x, d, axis=1)))
        if size < _F:
            x = x ^ (_flip_mask(size) ^ _flip_mask(2 * size))

    ones_f = (x & 1).astype(jnp.float32)
    perm = jax.lax.dot_general(ones_f, perm_ref[:, :],
                               (((1,), (0,)), ((), ())),
                               preferred_element_type=jnp.float32)
    ones = perm.astype(jnp.int32).reshape(_R // _NS, _NS, _F)
    out_ref[:, 0:_NS, :] = ones
    out_ref[:, _NS:2 * _NS, :] = 1 - ones


@functools.partial(jax.jit, static_argnums=())
def kernel(inputs):
    batch = inputs.shape[0]
    k_range = jnp.arange(1, _F, dtype=jnp.float32)
    w = 1.0 / (k_range * (_F - k_range))
    w = w / jnp.sum(w)
    logits = jnp.log(w)
    logits_p = jnp.concatenate(
        [logits, jnp.full((1,), -jnp.inf, jnp.float32)]).reshape(1, _F)

    pn = np.arange(_F)
    cn = (pn & 7) * 128 + (pn >> 3)
    P = np.zeros((_F, _F), np.float32)
    P[cn, pn] = 1.0
    perm = jnp.asarray(P)

    out = pl.pallas_call(
        _body,
        grid=(_G,),
        in_specs=[pl.BlockSpec((1, _F), lambda g: (0, 0)),
                  pl.BlockSpec((_F, _F), lambda g: (0, 0))],
        out_specs=pl.BlockSpec((_R // _NS, 2 * _NS, _F),
                               lambda g: (g, 0, 0)),
        out_shape=jax.ShapeDtypeStruct((batch, 2 * _NS, _F), jnp.int32),
        compiler_params=pltpu.CompilerParams(
            dimension_semantics=("arbitrary",)),
    )(logits_p, perm)
    return out


# speedup vs baseline: 1.0765x; 1.0765x over previous
import functools
import numpy as np
import jax
import jax.numpy as jnp
from jax import lax
from jax.experimental import pallas as pl
from jax.experimental.pallas import tpu as pltpu

_F = 1024
_NS = 8
_N = 1024 * _NS
_R = 128
_G = _N // _R

_U32 = np.uint32


def _np_rotl(x, r):
    return ((x << _U32(r)) | (x >> _U32(32 - r))).astype(_U32)


def _np_threefry(k1, k2, x0, x1):
    ks0, ks1 = _U32(k1), _U32(k2)
    ks2 = _U32(ks0 ^ ks1 ^ _U32(0x1BD11BDA))
    x0 = (x0 + ks0).astype(_U32)
    x1 = (x1 + ks1).astype(_U32)
    rot = ([13, 15, 26, 6], [17, 29, 16, 24])
    inj = [(ks1, ks2, 1), (ks2, ks0, 2), (ks0, ks1, 3), (ks1, ks2, 4),
           (ks2, ks0, 5)]
    for i in range(5):
        for r in rot[i % 2]:
            x0 = (x0 + x1).astype(_U32)
            x1 = _np_rotl(x1, r)
            x1 = (x1 ^ x0).astype(_U32)
        a, b, c = inj[i]
        x0 = (x0 + a).astype(_U32)
        x1 = (x1 + b + _U32(c)).astype(_U32)
    return x0, x1


_kc1, _kc2 = _np_threefry(0, 42, _U32(0), _U32(0))
_kp1, _kp2 = _np_threefry(0, 42, _U32(0), _U32(1))

_TINY = np.float32(np.finfo(np.float32).tiny)


def _const(v):
    return jnp.int32(np.uint32(v).view(np.int32))


def _rotl(x, r):
    return (x << r) | lax.shift_right_logical(x, 32 - r)


def _threefry(k1, k2, x0, x1):
    ks0 = _const(k1) if isinstance(k1, (int, np.integer)) else k1
    ks1 = _const(k2) if isinstance(k2, (int, np.integer)) else k2
    ks2 = ks0 ^ ks1 ^ _const(0x1BD11BDA)
    x0 = x0 + ks0
    x1 = x1 + ks1
    rot = ([13, 15, 26, 6], [17, 29, 16, 24])
    inj = [(ks1, ks2, 1), (ks2, ks0, 2), (ks0, ks1, 3), (ks1, ks2, 4),
           (ks2, ks0, 5)]
    for i in range(5):
        for r in rot[i % 2]:
            x0 = x0 + x1
            x1 = _rotl(x1, r)
            x1 = x1 ^ x0
        a, b, c = inj[i]
        x0 = x0 + a
        x1 = x1 + b + _const(c)
    return x0, x1


def _uniform_gumbel(bits):
    fb = lax.shift_right_logical(bits, 9) | _const(0x3F800000)
    f = lax.bitcast_convert_type(fb, jnp.float32) - jnp.float32(1.0)
    u = jnp.maximum(jnp.float32(_TINY), f + jnp.float32(_TINY))
    return -jnp.log(-jnp.log(u))


def _body(logits_ref, perm_ref, out_ref):
    g = pl.program_id(0)
    rows = g * _R + lax.broadcasted_iota(jnp.int32, (_R, 1), 0)

    c = lax.broadcasted_iota(jnp.int32, (_R, _F), 1)
    idx = rows * (_F - 1) + c
    y1, y2 = _threefry(int(_kc1), int(_kc2), jnp.zeros((_R, _F), jnp.int32), idx)
    gum = _uniform_gumbel(y1 ^ y2)
    score = gum + logits_ref[0, :][None, :]
    m = jnp.max(score, axis=1, keepdims=True)
    ni = jnp.min(jnp.where(score == m, c, _F), axis=1, keepdims=True)
    k = ni + 1

    z1 = jnp.zeros((_R, 1), jnp.int32)
    pk1, pk2 = _threefry(int(_kp1), int(_kp2), z1, rows)
    s1, s2 = _threefry(pk1, pk2, z1, z1 + 1)
    cphys = lax.broadcasted_iota(jnp.int32, (_R, _F), 1)
    jlab = ((cphys & 127) << 3) | lax.shift_right_logical(cphys, 7)
    b1, b2 = _threefry(jnp.broadcast_to(s1, (_R, _F)),
                       jnp.broadcast_to(s2, (_R, _F)),
                       jnp.zeros((_R, _F), jnp.int32), jlab)
    bits = b1 ^ b2
    payload = (jlab < k).astype(jnp.int32)
    key = ((bits ^ _const(0x80000000)) & _const(0xFFFFFFFE)) | payload

    crow = cphys[0:1, :]
    lrow = jlab[0:1, :]

    def _flip_mask(size):
        return 0 - ((lrow & size) != 0).astype(jnp.int32)

    x = key ^ _flip_mask(2)
    for size_log in range(1, 11):
        size = 1 << size_log
        for s_log in range(size_log - 1, -1, -1):
            s = 1 << s_log
            d = s * 128 if s < 8 else s // 8
            lower = (crow & d) == 0
            x = jnp.where(lower,
                          jnp.minimum(x, jnp.roll(x, -d, axis=1)),
                          jnp.maximum(x, jnp.roll(x, d, axis=1)))
        if size < _F:
            x = x ^ (_flip_mask(size) ^ _flip_mask(2 * size))

    ones_f = (x & 1).astype(jnp.float32)
    perm = jax.lax.dot_general(ones_f, perm_ref[:, :],
                               (((1,), (0,)), ((), ())),
                               preferred_element_type=jnp.float32)
    ones = perm.astype(jnp.int32).reshape(_R // _NS, _NS, _F)
    out_ref[:, 0:_NS, :] = ones
    out_ref[:, _NS:2 * _NS, :] = 1 - ones


@functools.partial(jax.jit, static_argnums=())
def kernel(inputs):
    batch = inputs.shape[0]
    k_range = jnp.arange(1, _F, dtype=jnp.float32)
    w = 1.0 / (k_range * (_F - k_range))
    w = w / jnp.sum(w)
    logits = jnp.log(w)
    logits_p = jnp.concatenate(
        [logits, jnp.full((1,), -jnp.inf, jnp.float32)]).reshape(1, _F)

    pn = np.arange(_F)
    cn = (pn & 7) * 128 + (pn >> 3)
    P = np.zeros((_F, _F), np.float32)
    P[cn, pn] = 1.0
    perm = jnp.asarray(P)

    out = pl.pallas_call(
        _body,
        grid=(_G,),
        in_specs=[pl.BlockSpec((1, _F), lambda g: (0, 0)),
                  pl.BlockSpec((_F, _F), lambda g: (0, 0))],
        out_specs=pl.BlockSpec((_R // _NS, 2 * _NS, _F),
                               lambda g: (g, 0, 0)),
        out_shape=jax.ShapeDtypeStruct((batch, 2 * _NS, _F), jnp.int32),
        compiler_params=pltpu.CompilerParams(
            dimension_semantics=("arbitrary",)),
    )(logits_p, perm)
    return out
